# Initial kernel scaffold; baseline (speedup 1.0000x reference)
import jax
import jax.numpy as jnp
from jax.experimental import pallas as pl


def kernel(x, ln1_g, ln1_b, qkv_w, qkv_b, out_w, out_b, ln2_g, ln2_b, conv_w, conv_b, ln3_g, ln3_b, gate_w, gate_b, expert_w, expert_b):
    raise NotImplementedError("write your pallas kernel here")


# speedup vs baseline: 2.9370x; 2.9370x over previous
import functools

import jax
import jax.numpy as jnp
from jax.experimental import pallas as pl
from jax.experimental.pallas import tpu as pltpu

B, L, C = 2, 2048, 1024
H = 16
DH = C // H
WIN = 256
E = 8
TOPK = 2
HID = 1024
ENTW = 0.1
LA = 1152
NWIN = 8
EPAD = 128

f32 = jnp.float32
bf16 = jnp.bfloat16


def _mm_nt(a, b):
    return jax.lax.dot_general(a.astype(bf16), b.astype(bf16),
                               (((1,), (1,)), ((), ())),
                               preferred_element_type=f32)


def _mm_tn(a, b):
    return jax.lax.dot_general(a.astype(bf16), b.astype(bf16),
                               (((0,), (0,)), ((), ())),
                               preferred_element_type=f32)


def _ln_rows(x, g, b):
    m = jnp.mean(x, axis=-1, keepdims=True)
    v = jnp.mean((x - m) ** 2, axis=-1, keepdims=True)
    return (x - m) * jax.lax.rsqrt(v + 1e-5) * g + b


def _qkv_kernel(x_ref, g_ref, b_ref, w_ref, wb_ref, o_ref):
    y = _ln_rows(x_ref[0], g_ref[...], b_ref[...])
    o_ref[0] = _mm_nt(y, w_ref[...]) + wb_ref[...]


def _qkv_call(x, g, b, w, wb):
    return pl.pallas_call(
        _qkv_kernel,
        grid=(B, 3),
        in_specs=[
            pl.BlockSpec((1, 384, C), lambda bb, i: (bb, i, 0)),
            pl.BlockSpec((1, C), lambda bb, i: (0, 0)),
            pl.BlockSpec((1, C), lambda bb, i: (0, 0)),
            pl.BlockSpec((3 * C, C), lambda bb, i: (0, 0)),
            pl.BlockSpec((1, 3 * C), lambda bb, i: (0, 0)),
        ],
        out_specs=pl.BlockSpec((1, 384, 3 * C), lambda bb, i: (bb, i, 0)),
        out_shape=jax.ShapeDtypeStruct((B, LA, 3 * C), f32),
    )(x, g, b, w, wb)


def _attn_kernel(q1_ref, q2_ref, o_ref):
    blk = jnp.concatenate([q1_ref[0], q2_ref[0]], axis=0)
    ri = jax.lax.broadcasted_iota(jnp.int32, (WIN, DH), 0)
    ci = jax.lax.broadcasted_iota(jnp.int32, (WIN, DH), 1)
    rexp = (ci == ri // 4).astype(f32)
    for h in range(H):
        q = blk[:, h * DH:(h + 1) * DH]
        k = blk[:, C + h * DH:C + (h + 1) * DH]
        v = blk[:, 2 * C + h * DH:2 * C + (h + 1) * DH]
        a = _mm_tn(q, k) * (DH ** -0.5)
        a = a - jnp.max(a, axis=-1, keepdims=True)
        p = jnp.exp(a)
        p = p / jnp.sum(p, axis=-1, keepdims=True)
        prep = jax.lax.dot_general(rexp.astype(bf16), p.astype(bf16),
                                   (((1,), (0,)), ((), ())),
                                   preferred_element_type=f32)
        ob = jnp.zeros((WIN, DH), f32)
        for tb in range(4):
            msk = (ri % 4 == tb).astype(f32)
            ob += _mm_nt(msk * prep, v[tb * DH:(tb + 1) * DH, :])
        o_ref[0, :, h * DH:(h + 1) * DH] = ob


def _attn_call(qkv):
    return pl.pallas_call(
        _attn_kernel,
        grid=(B, NWIN),
        in_specs=[
            pl.BlockSpec((1, 128, 3 * C), lambda bb, w: (bb, w, 0)),
            pl.BlockSpec((1, 128, 3 * C), lambda bb, w: (bb, w + 1, 0)),
        ],
        out_specs=pl.BlockSpec((1, WIN, C), lambda bb, w: (bb, w, 0)),
        out_shape=jax.ShapeDtypeStruct((B, L, C), f32),
    )(qkv, qkv)


def _proj_kernel(pre_ref, x_ref, w_ref, b_ref, o_ref):
    o_ref[0] = x_ref[0] + _mm_nt(pre_ref[0], w_ref[...]) + b_ref[...]


def _proj_call(pre, x, w, b):
    return pl.pallas_call(
        _proj_kernel,
        grid=(B, 4),
        in_specs=[
            pl.BlockSpec((1, 512, C), lambda bb, i: (bb, i, 0)),
            pl.BlockSpec((1, 512, C), lambda bb, i: (bb, i, 0)),
            pl.BlockSpec((C, C), lambda bb, i: (0, 0)),
            pl.BlockSpec((1, C), lambda bb, i: (0, 0)),
        ],
        out_specs=pl.BlockSpec((1, 512, C), lambda bb, i: (bb, i, 0)),
        out_shape=jax.ShapeDtypeStruct((B, L, C), f32),
    )(pre, x, w, b)


def _conv_kernel(hp_ref, hc_ref, hn_ref, g_ref, b_ref, w0_ref, w1_ref, w2_ref,
                 cb_ref, o_ref):
    i = pl.program_id(1)
    nb = pl.num_programs(1)
    g = g_ref[...]
    b = b_ref[...]
    zc = _ln_rows(hc_ref[0], g, b)
    zp = _ln_rows(hp_ref[0, -2:, :], g, b)
    zn = _ln_rows(hn_ref[0, :2, :], g, b)
    zp = jnp.where(i == 0, 0.0, zp)
    zn = jnp.where(i == nb - 1, 0.0, zn)
    zm2 = jnp.concatenate([zp, zc[:-2, :]], axis=0)
    zp2 = jnp.concatenate([zc[2:, :], zn], axis=0)
    y = _mm_nt(zm2, w0_ref[...]) + _mm_nt(zc, w1_ref[...]) \
        + _mm_nt(zp2, w2_ref[...]) + cb_ref[...]
    o_ref[0] = hc_ref[0] + y


def _conv_call(h1, g, b, w0, w1, w2, cb):
    row = pl.BlockSpec((1, 512, C), lambda bb, i: (bb, i, 0))
    return pl.pallas_call(
        _conv_kernel,
        grid=(B, 4),
        in_specs=[
            pl.BlockSpec((1, 512, C), lambda bb, i: (bb, jnp.maximum(i - 1, 0), 0)),
            row,
            pl.BlockSpec((1, 512, C), lambda bb, i: (bb, jnp.minimum(i + 1, 3), 0)),
            pl.BlockSpec((1, C), lambda bb, i: (0, 0)),
            pl.BlockSpec((1, C), lambda bb, i: (0, 0)),
            pl.BlockSpec((C, C), lambda bb, i: (0, 0)),
            pl.BlockSpec((C, C), lambda bb, i: (0, 0)),
            pl.BlockSpec((C, C), lambda bb, i: (0, 0)),
            pl.BlockSpec((1, C), lambda bb, i: (0, 0)),
        ],
        out_specs=row,
        out_shape=jax.ShapeDtypeStruct((B, L, C), f32),
    )(h1, h1, h1, g, b, w0, w1, w2, cb)


def _gate_kernel(h_ref, g_ref, b_ref, gw_ref, gb_ref,
                 aux_ref, z8_ref, wsel_ref, chosen_ref):
    i = pl.program_id(0)
    nb = pl.num_programs(0)
    z = _ln_rows(h_ref[...], g_ref[...], b_ref[...])
    logits = _mm_nt(z, gw_ref[...]) + gb_ref[...]
    col = jax.lax.broadcasted_iota(jnp.int32, logits.shape, 1)
    logits = jnp.where(col < E, logits, -1e30)
    logits = logits - jnp.max(logits, axis=-1, keepdims=True)
    ex = jnp.exp(logits)
    p = ex / jnp.sum(ex, axis=-1, keepdims=True)
    ent = -jnp.sum(jnp.sum(p * jnp.log(p + 1e-10), axis=1, keepdims=True),
                   axis=0, keepdims=True)

    @pl.when(i == 0)
    def _():
        aux_ref[...] = jnp.zeros((1, 1), f32)
        p8 = p[:E, :]
        c8 = jax.lax.broadcasted_iota(jnp.int32, p8.shape, 1)
        m1 = jnp.max(p8, axis=-1, keepdims=True)
        i1 = jnp.min(jnp.where(p8 == m1, c8, EPAD), axis=-1, keepdims=True)
        pm = jnp.where(c8 == i1, -1.0, p8)
        m2 = jnp.max(pm, axis=-1, keepdims=True)
        i2 = jnp.min(jnp.where(pm == m2, c8, EPAD), axis=-1, keepdims=True)
        rowi = jax.lax.broadcasted_iota(jnp.int32, (E, 1), 0)
        mask1 = i1 == rowi
        mask2 = i2 == rowi
        w1 = jnp.where(mask1, m1, 0.0)
        w2 = jnp.where(mask2, m2, 0.0)
        kkc = jax.lax.broadcasted_iota(jnp.int32, (E, EPAD), 1)
        wsel_ref[...] = jnp.where(kkc == 0, w1, jnp.where(kkc == 1, w2, 0.0))
        c1 = jnp.minimum(jnp.min(jnp.where(mask1, rowi, EPAD)), E - 1)
        c2 = jnp.minimum(jnp.min(jnp.where(mask2, rowi, EPAD)), E - 1)
        ci = jax.lax.broadcasted_iota(jnp.int32, (1, EPAD), 1)
        chosen_ref[...] = jnp.where(ci == 0, c1, jnp.where(ci == 1, c2, 0))
        z8_ref[...] = z[:E, :]

    aux_ref[...] += ent

    @pl.when(i == nb - 1)
    def _():
        aux_ref[...] *= ENTW / (B * L)


def _gate_call(h2f, g, b, gw_pad, gb_pad):
    return pl.pallas_call(
        _gate_kernel,
        grid=(8,),
        in_specs=[
            pl.BlockSpec((512, C), lambda i: (i, 0)),
            pl.BlockSpec((1, C), lambda i: (0, 0)),
            pl.BlockSpec((1, C), lambda i: (0, 0)),
            pl.BlockSpec((EPAD, C), lambda i: (0, 0)),
            pl.BlockSpec((1, EPAD), lambda i: (0, 0)),
        ],
        out_specs=[
            pl.BlockSpec((1, 1), lambda i: (0, 0)),
            pl.BlockSpec((E, C), lambda i: (0, 0)),
            pl.BlockSpec((E, EPAD), lambda i: (0, 0)),
            pl.BlockSpec((1, EPAD), lambda i: (0, 0)),
        ],
        out_shape=[
            jax.ShapeDtypeStruct((1, 1), f32),
            jax.ShapeDtypeStruct((E, C), f32),
            jax.ShapeDtypeStruct((E, EPAD), f32),
            jax.ShapeDtypeStruct((1, EPAD), jnp.int32),
        ],
    )(h2f, g, b, gw_pad, gb_pad)


def _expert_kernel(chosen_ref, z8_ref, wsel_ref, h8_ref, ew_ref, eb_ref, o_ref):
    kk = pl.program_id(0)
    y = _mm_nt(z8_ref[...], ew_ref[0]) + eb_ref[0]
    w = wsel_ref[...]
    col = jax.lax.broadcasted_iota(jnp.int32, w.shape, 1)
    scale = jnp.sum(jnp.where(col == kk, w, 0.0), axis=1, keepdims=True)
    contrib = scale * y

    @pl.when(kk == 0)
    def _():
        o_ref[...] = h8_ref[...] + contrib

    @pl.when(kk > 0)
    def _():
        o_ref[...] += contrib


def _expert_call(chosen, z8, wsel, h8, ew, eb):
    grid_spec = pltpu.PrefetchScalarGridSpec(
        num_scalar_prefetch=1,
        grid=(TOPK,),
        in_specs=[
            pl.BlockSpec((E, C), lambda kk, ch: (0, 0)),
            pl.BlockSpec((E, EPAD), lambda kk, ch: (0, 0)),
            pl.BlockSpec((E, HID), lambda kk, ch: (0, 0)),
            pl.BlockSpec((1, HID, C), lambda kk, ch: (ch[kk], 0, 0)),
            pl.BlockSpec((1, 1, HID), lambda kk, ch: (ch[kk], 0, 0)),
        ],
        out_specs=pl.BlockSpec((E, HID), lambda kk, ch: (0, 0)),
    )
    return pl.pallas_call(
        _expert_kernel,
        grid_spec=grid_spec,
        out_shape=jax.ShapeDtypeStruct((E, HID), f32),
    )(chosen, z8, wsel, h8, ew, eb)


@jax.jit
def kernel(x, ln1_g, ln1_b, qkv_w, qkv_b, out_w, out_b, ln2_g, ln2_b,
           conv_w, conv_b, ln3_g, ln3_b, gate_w, gate_b, expert_w, expert_b):
    r1 = lambda a: a.reshape(1, -1)
    qkv = _qkv_call(x, r1(ln1_g), r1(ln1_b), qkv_w, r1(qkv_b))
    pre = _attn_call(qkv)
    h1 = _proj_call(pre, x, out_w, r1(out_b))
    h2 = _conv_call(h1, r1(ln2_g), r1(ln2_b),
                    conv_w[:, :, 0], conv_w[:, :, 1], conv_w[:, :, 2],
                    r1(conv_b))
    gw_pad = jnp.zeros((EPAD, C), f32).at[:E].set(gate_w)
    gb_pad = jnp.zeros((1, EPAD), f32).at[0, :E].set(gate_b)
    aux_arr, z8, wsel, chosen = _gate_call(h2.reshape(B * L, C),
                                           r1(ln3_g), r1(ln3_b), gw_pad, gb_pad)
    out8 = _expert_call(chosen[0, :TOPK], z8, wsel, h2[0, :E], expert_w,
                        expert_b.reshape(E, 1, HID))
    out = jax.lax.dynamic_update_slice(h2, out8[None], (0, 0, 0))
    return out, aux_arr[0, 0]
